# baseline (speedup 1.0000x reference)
import functools

import jax
import jax.numpy as jnp
from jax import lax
from jax.experimental import pallas as pl
from jax.experimental.pallas import tpu as pltpu
from jax.experimental.pallas import tpu_sc as plsc

_G = 76
_GG = _G * _G
_NUM = 3
_CP = 14
_NCH = _NUM * _CP
_NCLS = 7
_NW = 32
_NC = 2
_CHUNK = 304
_NCHK = _GG // _CHUNK
_GRP = _CHUNK // 16
_ROW15 = _CHUNK * 15


def _sig(v):
    return 1.0 / (1.0 + jnp.exp(-v))


def _atan(z):
    az = jnp.abs(z)
    inv = az > 1.0
    u = jnp.where(inv, 1.0 / az, az)
    u2 = u * u
    p = u * (0.9998660 + u2 * (-0.3302995 + u2 * (
        0.1801410 + u2 * (-0.0851330 + u2 * 0.0208351))))
    r = jnp.where(inv, (jnp.pi / 2.0) - p, p)
    return jnp.sign(z) * r


def _make_sc(B):
    n_items = B * _NCHK
    n_iter = (n_items + _NW - 1) // _NW
    mesh = plsc.VectorSubcoreMesh(core_axis_name="c", subcore_axis_name="s")

    @functools.partial(
        pl.kernel,
        out_type=jax.ShapeDtypeStruct((B * _NUM * _GG * 15,), jnp.float32),
        mesh=mesh,
        compiler_params=pltpu.CompilerParams(needs_layout_passes=False),
        scratch_types=[
            pltpu.VMEM((_NCH * _CHUNK,), jnp.float32),
            pltpu.VMEM((_NUM * _ROW15,), jnp.float32),
            pltpu.VMEM((_CHUNK,), jnp.float32),
            pltpu.VMEM((_CHUNK,), jnp.float32),
            pltpu.VMEM((_NUM * 32,), jnp.float32),
            pltpu.VMEM((_CHUNK,), jnp.int32),
            pltpu.SemaphoreType.DMA,
            pltpu.SemaphoreType.DMA,
        ],
    )
    def sck(x_hbm, gx_hbm, gy_hbm, asp_hbm, idx_hbm, out_hbm,
            slab, stage, gxv, gyv, aspv, idxv, sem_in, sem_out):
        wid = lax.axis_index("s") * _NC + lax.axis_index("c")
        pltpu.sync_copy(asp_hbm, aspv)
        pltpu.sync_copy(idx_hbm, idxv)
        for it in range(n_iter):
            item = wid + _NW * it

            @pl.when(item < n_items)
            def _process(item=item):
                b = item // _NCHK
                ch = item - b * _NCHK
                start = ch * _CHUNK
                copies = [
                    pltpu.async_copy(gx_hbm.at[pl.ds(start, _CHUNK)],
                                     gxv, sem_in),
                    pltpu.async_copy(gy_hbm.at[pl.ds(start, _CHUNK)],
                                     gyv, sem_in),
                ]
                for c in range(_NCH):
                    copies.append(pltpu.async_copy(
                        x_hbm.at[pl.ds((b * _NCH + c) * _GG + start, _CHUNK)],
                        slab.at[pl.ds(c * _CHUNK, _CHUNK)],
                        sem_in))
                for cp in copies:
                    cp.wait()
                for a in range(_NUM):
                    c0 = a * _CP
                    crows = [((3 * k + a) // _NCLS) * _CP + _NCLS
                             + (3 * k + a) % _NCLS for k in range(_NCLS)]
                    @pl.loop(0, _GRP)
                    def _grp(g, a=a, c0=c0, crows=crows):
                        awv = aspv[pl.ds(a * 32, 16)]
                        ahv = aspv[pl.ds(a * 32 + 16, 16)]
                        def row(c):
                            return slab[pl.ds(c * _CHUNK + g * 16, 16)]

                        sl = pl.ds(g * 16, 16)
                        gx = gxv[sl]
                        gy = gyv[sl]
                        im = row(c0 + 4)
                        re_ = row(c0 + 5)
                        yaw = _atan(im / re_)
                        conf = _sig(row(c0 + 6))
                        ax = ((_sig(row(c0 + 0)) + gx)
                              * 8.0).astype(jnp.int32).astype(jnp.float32)
                        ay = ((_sig(row(c0 + 1)) + gy)
                              * 8.0).astype(jnp.int32).astype(jnp.float32)
                        aw = jnp.exp(row(c0 + 2)) * awv
                        ah = jnp.exp(row(c0 + 3)) * ahv
                        vals = [im, re_, yaw, conf, ax, ay, aw, ah]
                        for k in range(_NCLS):
                            vals.append(row(crows[k]))
                        biv = idxv[sl]
                        for fidx, v in enumerate(vals):
                            plsc.store_scatter(
                                stage, [biv + (a * _ROW15 + fidx)], v)
                obase = (b * _NUM * _GG + ch * _CHUNK) * 15
                outs = []
                for a in range(_NUM):
                    outs.append(pltpu.async_copy(
                        stage.at[pl.ds(a * _ROW15, _ROW15)],
                        out_hbm.at[pl.ds(obase + a * _GG * 15, _ROW15)],
                        sem_out))
                for cp in outs:
                    cp.wait()

    return sck


def kernel(x, anchors):
    B = x.shape[0]
    xr = x.reshape(-1)
    pp = jnp.arange(_GG, dtype=jnp.int32)
    gxt = (pp % _G).astype(jnp.float32)
    gyt = (pp // _G).astype(jnp.float32)
    asp = jnp.repeat(anchors.reshape(_NUM * 2), 16).reshape(_NUM, 2 * 16)
    asp = asp.reshape(_NUM * 32)
    idxt = jnp.arange(_CHUNK, dtype=jnp.int32) * 15
    out = _make_sc(B)(xr, gxt, gyt, asp, idxt)
    return out.reshape(B, _NUM * _GG, 15)
